# Initial kernel scaffold; baseline (speedup 1.0000x reference)
import jax
import jax.numpy as jnp
from jax.experimental import pallas as pl


def kernel(images, camera_projection, t_lidar_camera, params):
    raise NotImplementedError("write your pallas kernel here")


# speedup vs baseline: 1.6234x; 1.6234x over previous
import functools
import math

import jax
import jax.numpy as jnp
import numpy as np
from jax.experimental import pallas as pl
from jax.experimental.pallas import tpu as pltpu

_IMG_H = 512
_IMG_W = 512
_C_OUT = 128
_NY = 200
_NX = 200
_DB = 64
_DEPTH_MIN = 1.0
_DEPTH_MAX = 60.0
_SIGMA = 0.8
_MIN_OP = 0.05
_EPS = 1e-6
_PC = np.array([-51.2, -51.2, -20.0, 51.2, 51.2, 20.0], dtype=np.float32)
_VS = np.array([0.512, 0.512, 40.0], dtype=np.float32)
_PAD = 2
_NYP = _NY + 2 * _PAD
_NXP = _NX + 2 * _PAD
_TILES = 8
_ROWS = _NY // _TILES


def _gauss_kw():
    sig2 = _SIGMA * _SIGMA
    k = np.zeros((3, 3), np.float32)
    for dy in (-1, 0, 1):
        for dx in (-1, 0, 1):
            k[dy + 1, dx + 1] = np.float32(float(np.exp(-(dx * dx + dy * dy) / (2.0 * sig2))))
    return k


def _conv(x, w, stride, pad):
    return jax.lax.conv_general_dilated(
        x, w, (stride, stride), [(pad, pad), (pad, pad)],
        dimension_numbers=('NCHW', 'OIHW', 'NCHW'))


def _bn(x, g, b, m, v):
    return (x - m[None, :, None, None]) / jnp.sqrt(v[None, :, None, None] + 1e-5) * g[None, :, None, None] + b[None, :, None, None]


def _scatter_kernel(py_ref, px_ref, wv_ref, pf_ref, out_ref, canvas_ref, wacc_ref):
    t = pl.program_id(1)
    dyg = jax.lax.broadcasted_iota(jnp.int32, (3, 3), 0).astype(jnp.float32) - 1.0
    dxg = jax.lax.broadcasted_iota(jnp.int32, (3, 3), 1).astype(jnp.float32) - 1.0
    kw = jnp.exp(-(dxg * dxg + dyg * dyg) / (2.0 * _SIGMA * _SIGMA))

    @pl.when(t == 0)
    def _():
        canvas_ref[...] = jnp.zeros(canvas_ref.shape, canvas_ref.dtype)
        wacc_ref[...] = jnp.zeros(wacc_ref.shape, wacc_ref.dtype)

        def body(j, carry):
            w = wv_ref[0, 0, j]
            py = py_ref[0, 0, j]
            px = px_ref[0, 0, j]
            f = pf_ref[0, j, :]
            kww = kw * w
            patch = canvas_ref[pl.ds(py, 3), pl.ds(px, 3), :]
            canvas_ref[pl.ds(py, 3), pl.ds(px, 3), :] = patch + kw[:, :, None] * f[None, None, :]
            wp = wacc_ref[pl.ds(py, 3), pl.ds(px, 3), :]
            wacc_ref[pl.ds(py, 3), pl.ds(px, 3), :] = wp + kww[:, :, None]
            return carry

        jax.lax.fori_loop(0, py_ref.shape[2], body, 0)

    y0 = t * _ROWS + _PAD
    c = canvas_ref[pl.ds(y0, _ROWS), _PAD:_PAD + _NX, :]
    wv = wacc_ref[pl.ds(y0, _ROWS), _PAD:_PAD + _NX, 0:1]
    out_ref[0] = c / jnp.maximum(wv, _EPS)


def _bev_scatter(py, px, wv, pf, B, N):
    return pl.pallas_call(
        _scatter_kernel,
        grid=(B, _TILES),
        in_specs=[
            pl.BlockSpec((1, 1, N), lambda b, t: (b, 0, 0), memory_space=pltpu.SMEM),
            pl.BlockSpec((1, 1, N), lambda b, t: (b, 0, 0), memory_space=pltpu.SMEM),
            pl.BlockSpec((1, 1, N), lambda b, t: (b, 0, 0), memory_space=pltpu.SMEM),
            pl.BlockSpec((1, N, _C_OUT), lambda b, t: (b, 0, 0)),
        ],
        out_specs=pl.BlockSpec((1, _ROWS, _NX, _C_OUT), lambda b, t: (b, t, 0, 0)),
        out_shape=jax.ShapeDtypeStruct((B, _NY, _NX, _C_OUT), jnp.float32),
        scratch_shapes=[
            pltpu.VMEM((_NYP, _NXP, _C_OUT), jnp.float32),
            pltpu.VMEM((_NYP, _NXP, 8), jnp.float32),
        ],
    )(py.reshape(B, 1, N), px.reshape(B, 1, N), wv.reshape(B, 1, N), pf)


def kernel(images, camera_projection, t_lidar_camera, params):
    p = params
    x = images
    for i in range(1, 5):
        x = _conv(x, p['enc_w%d' % i], 2, 1)
        x = _bn(x, p['enc_g%d' % i], p['enc_b%d' % i], p['enc_m%d' % i], p['enc_v%d' % i])
        x = jnp.maximum(x, 0.0)
    feats = x
    f = _conv(feats, p['fh_w1'], 1, 1)
    f = jnp.maximum(_bn(f, p['fh_g1'], p['fh_b1'], p['fh_m1'], p['fh_v1']), 0.0)
    features = _conv(f, p['fh_w2'], 1, 0) + p['fh_bias2'][None, :, None, None]
    depth_logits = _conv(feats, p['dh_w'], 1, 0) + p['dh_bias'][None, :, None, None]
    depth_probs = jax.nn.softmax(depth_logits, axis=1)
    opacity = jax.nn.sigmoid(_conv(feats, p['oh_w'], 1, 0) + p['oh_bias'][None, :, None, None])[:, 0]
    B, C, Hf, Wf = features.shape
    depth_values = jnp.linspace(_DEPTH_MIN, _DEPTH_MAX, _DB).astype(jnp.float32)
    z = jnp.sum(depth_probs * depth_values[None, :, None, None], axis=1)[:, None]
    ys = (jnp.arange(Hf, dtype=jnp.float32) + 0.5) * (float(_IMG_H) / float(Hf))
    xs = (jnp.arange(Wf, dtype=jnp.float32) + 0.5) * (float(_IMG_W) / float(Wf))
    yy, xx = jnp.meshgrid(ys, xs, indexing='ij')
    yy = yy[None, None]
    xx = xx[None, None]
    fx = jnp.clip(camera_projection[:, 0, 0], _EPS, None).reshape(B, 1, 1, 1)
    fy = jnp.clip(camera_projection[:, 1, 1], _EPS, None).reshape(B, 1, 1, 1)
    cx = camera_projection[:, 0, 2].reshape(B, 1, 1, 1)
    cy = camera_projection[:, 1, 2].reshape(B, 1, 1, 1)
    x_cam = (xx - cx) * z / fx
    y_cam = (yy - cy) * z / fy
    cam_homo = jnp.stack([x_cam, y_cam, z, jnp.ones_like(z)], axis=-1).reshape(B, -1, 4)
    lidar = jnp.einsum('bij,bnj->bni', t_lidar_camera, cam_homo)[..., :3]
    pc = jnp.asarray(_PC)
    vs = jnp.asarray(_VS)
    x_i = jnp.floor((lidar[..., 0] - pc[0]) / vs[0]).astype(jnp.int32)
    y_i = jnp.floor((lidar[..., 1] - pc[1]) / vs[1]).astype(jnp.int32)
    z_ok = (lidar[..., 2] >= pc[2]) & (lidar[..., 2] < pc[5])
    N = Hf * Wf
    feat_bnc = features.transpose(0, 2, 3, 1).reshape(B, N, C)
    op_b = opacity.reshape(B, N)
    base_valid = (op_b >= _MIN_OP) & z_ok
    base_w = op_b * base_valid.astype(jnp.float32)
    ok = base_valid & (x_i >= -1) & (x_i <= _NX) & (y_i >= -1) & (y_i <= _NY)
    wv = jnp.where(ok, base_w, 0.0)
    px = jnp.where(ok, x_i + 1, 0).astype(jnp.int32)
    py = jnp.where(ok, y_i + 1, 0).astype(jnp.int32)
    pf = feat_bnc * wv[:, :, None]
    bev = _bev_scatter(py, px, wv, pf, B, N)
    return bev.transpose(0, 3, 1, 2)
